# Initial kernel scaffold; baseline (speedup 1.0000x reference)
import jax
import jax.numpy as jnp
from jax.experimental import pallas as pl


def kernel(x, edge_index, Wl0, bl0, Wr0, Wl1, bl1, Wr1, Wl2, bl2, Wr2, g0, beta0, g1, beta1, Wp, bp):
    raise NotImplementedError("write your pallas kernel here")


# speedup vs baseline: 4.2431x; 4.2431x over previous
import functools

import jax
import jax.numpy as jnp
from jax import lax
from jax.experimental import pallas as pl
from jax.experimental.pallas import tpu as pltpu
from jax.experimental.pallas import tpu_sc as plsc

_NC = 2
_NS = 16
_NW = _NC * _NS


def _pad_n(n):
    return ((n + 127) // 128) * 128


@functools.partial(jax.jit, static_argnames=("n", "d", "e"))
def _sc_segment_sum(h, src, dst, zeros_tile, *, n, d, e):
    epw = e // _NW
    c_sz = 80
    n_chunks = epw // c_sz
    np_ = _pad_n(n)
    rpt = np_ // _NS
    assert epw % c_sz == 0 and e % _NW == 0

    mesh = plsc.VectorSubcoreMesh(core_axis_name="c", subcore_axis_name="s")

    @functools.partial(
        pl.kernel,
        mesh=mesh,
        out_type=jax.ShapeDtypeStruct((_NC * np_, d), jnp.float32),
        scratch_types=[
            pltpu.VMEM((c_sz,), jnp.int32),
            pltpu.VMEM((c_sz,), jnp.int32),
            pltpu.VMEM((c_sz, d), jnp.float32),
            pltpu.VMEM_SHARED((np_, d), jnp.float32),
            pltpu.SemaphoreType.DMA,
        ],
    )
    def agg_kernel(h_hbm, src_hbm, dst_hbm, z_hbm, out_hbm,
                   src_v, dst_v, rows_v, acc_sh, sem):
        c = lax.axis_index("c")
        s = lax.axis_index("s")
        wid = s * _NC + c
        r0 = s * rpt
        pltpu.sync_copy(z_hbm, acc_sh.at[pl.ds(r0, rpt)])
        plsc.subcore_barrier()

        base = wid * epw

        def body(i, carry):
            off = base + i * c_sz
            pltpu.sync_copy(src_hbm.at[pl.ds(off, c_sz)], src_v)
            pltpu.sync_copy(dst_hbm.at[pl.ds(off, c_sz)], dst_v)
            pltpu.async_copy(h_hbm.at[src_v], rows_v, sem).wait()
            pltpu.sync_copy(rows_v, acc_sh.at[dst_v], add=True)
            return carry

        lax.fori_loop(0, n_chunks, body, 0)
        plsc.subcore_barrier()
        pltpu.sync_copy(acc_sh.at[pl.ds(r0, rpt)],
                        out_hbm.at[pl.ds(c * np_ + r0, rpt)])

    out = agg_kernel(h, src, dst, zeros_tile)
    return out.reshape(_NC, np_, d)[:, :n, :]


@functools.partial(jax.jit, static_argnames=("n", "d", "e"))
def _sc_degree(dst, ones_tile, zeros_tile, *, n, d, e):
    epw = e // _NW
    c_sz = 80
    n_chunks = epw // c_sz
    np_ = _pad_n(n)
    rpt = np_ // _NS

    mesh = plsc.VectorSubcoreMesh(core_axis_name="c", subcore_axis_name="s")

    @functools.partial(
        pl.kernel,
        mesh=mesh,
        out_type=jax.ShapeDtypeStruct((_NC * np_, d), jnp.float32),
        scratch_types=[
            pltpu.VMEM((c_sz,), jnp.int32),
            pltpu.VMEM((c_sz, d), jnp.float32),
            pltpu.VMEM_SHARED((np_, d), jnp.float32),
        ],
    )
    def deg_kernel(dst_hbm, ones_hbm, z_hbm, out_hbm, dst_v, ones_v, deg_sh):
        c = lax.axis_index("c")
        s = lax.axis_index("s")
        wid = s * _NC + c
        r0 = s * rpt
        pltpu.sync_copy(z_hbm, deg_sh.at[pl.ds(r0, rpt)])
        pltpu.sync_copy(ones_hbm, ones_v)
        plsc.subcore_barrier()

        base = wid * epw

        def body(i, carry):
            off = base + i * c_sz
            pltpu.sync_copy(dst_hbm.at[pl.ds(off, c_sz)], dst_v)
            pltpu.sync_copy(ones_v, deg_sh.at[dst_v], add=True)
            return carry

        lax.fori_loop(0, n_chunks, body, 0)
        plsc.subcore_barrier()
        pltpu.sync_copy(deg_sh.at[pl.ds(r0, rpt)],
                        out_hbm.at[pl.ds(c * np_ + r0, rpt)])

    out = deg_kernel(dst, ones_tile, zeros_tile)
    return out.reshape(_NC, np_, d)[:, :n, :]


def _tc_layer_bn_relu(P, degp, h, Wl, bl, Wr, g, beta):
    n, d = h.shape

    def body(p_ref, degp_ref, h_ref, wl_ref, bl_ref, wr_ref, g_ref, b_ref,
             o_ref):
        deg = degp_ref[0, :, 0:1] + degp_ref[1, :, 0:1]
        inv = 1.0 / jnp.maximum(deg, 1.0)
        mean = (p_ref[0] + p_ref[1]) * inv
        z = (jnp.dot(mean, wl_ref[...], preferred_element_type=jnp.float32)
             + bl_ref[...]
             + jnp.dot(h_ref[...], wr_ref[...],
                       preferred_element_type=jnp.float32))
        mu = jnp.mean(z, axis=0, keepdims=True)
        var = jnp.mean((z - mu) * (z - mu), axis=0, keepdims=True)
        zn = g_ref[...] * (z - mu) * lax.rsqrt(var + 1e-5) + b_ref[...]
        o_ref[...] = jnp.maximum(zn, 0.0)

    return pl.pallas_call(
        body,
        out_shape=jax.ShapeDtypeStruct((n, d), jnp.float32),
    )(P, degp, h, Wl, bl.reshape(1, -1), Wr, g.reshape(1, -1),
      beta.reshape(1, -1))


def _tc_layer_final(P, degp, h, Wl, bl, Wr, Wp, bp):
    n, d = h.shape
    o = Wp.shape[1]

    def body(p_ref, degp_ref, h_ref, wl_ref, bl_ref, wr_ref, wp_ref, bp_ref,
             out_ref):
        deg = degp_ref[0, :, 0:1] + degp_ref[1, :, 0:1]
        inv = 1.0 / jnp.maximum(deg, 1.0)
        mean = (p_ref[0] + p_ref[1]) * inv
        z = (jnp.dot(mean, wl_ref[...], preferred_element_type=jnp.float32)
             + bl_ref[...]
             + jnp.dot(h_ref[...], wr_ref[...],
                       preferred_element_type=jnp.float32))
        logits = jnp.dot(z, wp_ref[...],
                         preferred_element_type=jnp.float32) + bp_ref[...]
        m = jnp.max(logits, axis=1, keepdims=True)
        shifted = logits - m
        lse = jnp.log(jnp.sum(jnp.exp(shifted), axis=1, keepdims=True))
        out_ref[...] = shifted - lse

    return pl.pallas_call(
        body,
        out_shape=jax.ShapeDtypeStruct((n, o), jnp.float32),
    )(P, degp, h, Wl, bl.reshape(1, -1), Wr, Wp, bp.reshape(1, -1))


def kernel(x, edge_index, Wl0, bl0, Wr0, Wl1, bl1, Wr1, Wl2, bl2, Wr2,
           g0, beta0, g1, beta1, Wp, bp):
    n, d = x.shape
    e = edge_index.shape[1]
    src = edge_index[0]
    dst = edge_index[1]

    rpt = _pad_n(n) // _NS
    zeros_tile = jnp.zeros((rpt, d), jnp.float32)
    ones_tile = jnp.ones((80, d), jnp.float32)

    degp = _sc_degree(dst, ones_tile, zeros_tile, n=n, d=d, e=e)

    P0 = _sc_segment_sum(x, src, dst, zeros_tile, n=n, d=d, e=e)
    h1 = _tc_layer_bn_relu(P0, degp, x, Wl0, bl0, Wr0, g0, beta0)

    P1 = _sc_segment_sum(h1, src, dst, zeros_tile, n=n, d=d, e=e)
    h2 = _tc_layer_bn_relu(P1, degp, h1, Wl1, bl1, Wr1, g1, beta1)

    P2 = _sc_segment_sum(h2, src, dst, zeros_tile, n=n, d=d, e=e)
    return _tc_layer_final(P2, degp, h2, Wl2, bl2, Wr2, Wp, bp)
